# baseline (speedup 1.0000x reference)
import functools

import jax
import jax.numpy as jnp
from jax import lax
from jax.experimental import pallas as pl
from jax.experimental.pallas import tpu as pltpu
from jax.experimental.pallas import tpu_sc as plsc

N = 10000
D = 128
E = 320000
NCORE = 2
NSUB = 16
NT = NCORE * NSUB
EPT = E // NT
K = 125
NCHT = 80
EPAD = NCHT * K - EPT
UNROLL = 16
NBLK = NCHT // UNROLL
NPAD = 10240
RPT = NPAD // NSUB
TRASH = N

_mesh = plsc.VectorSubcoreMesh(core_axis_name="c", subcore_axis_name="s")


def _deg_body(dst_hbm, zeros_hbm, ones_hbm, out_hbm, dst_v, ones_v, acc_sh):
    c = lax.axis_index("c")
    s = lax.axis_index("s")
    tile = c * NSUB + s
    pltpu.sync_copy(dst_hbm.at[tile], dst_v)
    pltpu.sync_copy(ones_hbm, ones_v)
    pltpu.sync_copy(zeros_hbm, acc_sh.at[pl.ds(s * RPT, RPT)])
    plsc.subcore_barrier()

    def body(j, carry):
        pltpu.sync_copy(ones_v, acc_sh.at[dst_v.at[j]], add=True)
        return carry

    lax.fori_loop(0, NCHT, body, 0)
    plsc.subcore_barrier()
    pltpu.sync_copy(acc_sh.at[pl.ds(s * RPT, RPT)],
                    out_hbm.at[c, pl.ds(s * RPT, RPT)])


_deg_call = functools.partial(
    pl.kernel,
    out_type=jax.ShapeDtypeStruct((NCORE, NPAD, D), jnp.float32),
    mesh=_mesh,
    scratch_types=[
        pltpu.VMEM((NCHT, K), jnp.int32),
        pltpu.VMEM((K, D), jnp.float32),
        pltpu.VMEM_SHARED((NPAD, D), jnp.float32),
    ],
)(_deg_body)


def _agg_body(hs_hbm, src_hbm, dst_hbm, zeros_hbm, out_hbm,
              src_v, dst_v, rows0, rows1, acc_sh, sem0, sem1):
    c = lax.axis_index("c")
    s = lax.axis_index("s")
    tile = c * NSUB + s
    pltpu.sync_copy(zeros_hbm, acc_sh.at[pl.ds(s * RPT, RPT)])
    plsc.subcore_barrier()

    rows = (rows0, rows1)
    sems = (sem0, sem1)

    def block(b, carry):
        pltpu.sync_copy(src_hbm.at[tile, pl.ds(b * UNROLL, UNROLL)], src_v)
        pltpu.sync_copy(dst_hbm.at[tile, pl.ds(b * UNROLL, UNROLL)], dst_v)
        g_prev = pltpu.async_copy(hs_hbm.at[src_v.at[0]], rows[0], sems[0])
        for i in range(1, UNROLL):
            g_cur = pltpu.async_copy(hs_hbm.at[src_v.at[i]], rows[i % 2],
                                     sems[i % 2])
            g_prev.wait()
            pltpu.sync_copy(rows[(i - 1) % 2], acc_sh.at[dst_v.at[i - 1]],
                            add=True)
            g_prev = g_cur
        g_prev.wait()
        pltpu.sync_copy(rows[(UNROLL - 1) % 2],
                        acc_sh.at[dst_v.at[UNROLL - 1]], add=True)
        return carry

    lax.fori_loop(0, NBLK, block, 0)
    plsc.subcore_barrier()
    pltpu.sync_copy(acc_sh.at[pl.ds(s * RPT, RPT)],
                    out_hbm.at[c, pl.ds(s * RPT, RPT)])


_agg_call = functools.partial(
    pl.kernel,
    out_type=jax.ShapeDtypeStruct((NCORE, NPAD, D), jnp.float32),
    mesh=_mesh,
    scratch_types=[
        pltpu.VMEM((UNROLL, K), jnp.int32),
        pltpu.VMEM((UNROLL, K), jnp.int32),
        pltpu.VMEM((K, D), jnp.float32),
        pltpu.VMEM((K, D), jnp.float32),
        pltpu.VMEM_SHARED((NPAD, D), jnp.float32),
        pltpu.SemaphoreType.DMA,
        pltpu.SemaphoreType.DMA,
    ],
)(_agg_body)


_BLK = 2000


def _dis_block(degp):
    deg = degp[0] + degp[1]
    return lax.rsqrt(deg[:, 0:1] + 1.0)


def _tc_a_body(x_ref, wct_ref, wrt_ref, bc_ref, br_ref, h_ref, pre_ref):
    X = jnp.tanh(x_ref[...])
    h = jnp.dot(X, wct_ref[...], preferred_element_type=jnp.float32)
    r = jnp.dot(h, wrt_ref[...], preferred_element_type=jnp.float32)
    h_ref[...] = h
    pre_ref[...] = bc_ref[...] - r - br_ref[...]


def _tc_b_body(h_ref, pre_ref, degp_ref, hs_ref, base_ref):
    dis = _dis_block(degp_ref[...])
    h = h_ref[...]
    hs = h * dis
    hs_ref[...] = hs
    base_ref[...] = hs * dis + pre_ref[...]


def _tc_hs_body(sp_ref, basep_ref, degp_ref, wct_ref, hs_ref, h_ref):
    dis = _dis_block(degp_ref[...])
    S = sp_ref[0] + sp_ref[1]
    X = jnp.tanh(S * dis + basep_ref[...])
    h = jnp.dot(X, wct_ref[...], preferred_element_type=jnp.float32)
    h_ref[...] = h
    hs_ref[...] = h * dis


def _tc_base_body(h_ref, hs_ref, degp_ref, wrt_ref, bc_ref, br_ref, base_ref):
    dis = _dis_block(degp_ref[...])
    r = jnp.dot(h_ref[...], wrt_ref[...], preferred_element_type=jnp.float32)
    base_ref[...] = hs_ref[...] * dis + bc_ref[...] - r - br_ref[...]


def _tc_final_body(sp_ref, basep_ref, degp_ref, x_ref):
    dis = _dis_block(degp_ref[...])
    S = sp_ref[0] + sp_ref[1]
    x_ref[...] = jnp.tanh(S * dis + basep_ref[...])


_io_nd = lambda: pl.BlockSpec((_BLK, D), lambda i: (i, 0))
_io_degp = lambda: pl.BlockSpec((NCORE, _BLK, D), lambda i: (0, i, 0))
_io_sp = lambda: pl.BlockSpec((NCORE, _BLK, D), lambda i: (0, i, 0))
_io_w = lambda: pl.BlockSpec((D, D), lambda i: (0, 0))
_io_b = lambda: pl.BlockSpec((1, D), lambda i: (0, 0))

_nd_out = lambda n: [jax.ShapeDtypeStruct((N, D), jnp.float32)] * n

_tc_a = pl.pallas_call(
    _tc_a_body,
    grid=(N // _BLK,),
    in_specs=[_io_nd(), _io_w(), _io_w(), _io_b(), _io_b()],
    out_specs=[_io_nd(), _io_nd()],
    out_shape=_nd_out(2),
)

_tc_b = pl.pallas_call(
    _tc_b_body,
    grid=(N // _BLK,),
    in_specs=[_io_nd(), _io_nd(), _io_degp()],
    out_specs=[_io_nd(), _io_nd()],
    out_shape=_nd_out(2),
)

_tc_hs = pl.pallas_call(
    _tc_hs_body,
    grid=(N // _BLK,),
    in_specs=[_io_sp(), _io_nd(), _io_degp(), _io_w()],
    out_specs=[_io_nd(), _io_nd()],
    out_shape=_nd_out(2),
)

_tc_base = pl.pallas_call(
    _tc_base_body,
    grid=(N // _BLK,),
    in_specs=[_io_nd(), _io_nd(), _io_degp(), _io_w(), _io_b(), _io_b()],
    out_specs=_io_nd(),
    out_shape=_nd_out(1)[0],
)

_tc_final = pl.pallas_call(
    _tc_final_body,
    grid=(N // _BLK,),
    in_specs=[_io_sp(), _io_nd(), _io_degp()],
    out_specs=_io_nd(),
    out_shape=jax.ShapeDtypeStruct((N, D), jnp.float32),
)


def kernel(x, edge_index, W_conv, b_conv, W_res, b_res):
    src = edge_index[0].astype(jnp.int32).reshape(NT, EPT)
    dst = edge_index[1].astype(jnp.int32).reshape(NT, EPT)
    srcp = jnp.pad(src, ((0, 0), (0, EPAD))).reshape(NT, NCHT, K)
    dstp = jnp.pad(dst, ((0, 0), (0, EPAD)),
                   constant_values=TRASH).reshape(NT, NCHT, K)
    wct = W_conv.T
    wrt = W_res.T
    bc = b_conv.reshape(1, D)
    br = b_res.reshape(1, D)
    ones_deg = jnp.ones((K, D), jnp.float32)
    zeros_rows = jnp.zeros((RPT, D), jnp.float32)

    degp = _deg_call(dstp, zeros_rows, ones_deg)
    h, pre = _tc_a(x, wct, wrt, bc, br)
    hs, base = _tc_b(h, pre, degp)
    for _ in range(2):
        sp = _agg_call(hs, srcp, dstp, zeros_rows)
        hs, h = _tc_hs(sp, base, degp, wct)
        base = _tc_base(h, hs, degp, wrt, bc, br)
    sp = _agg_call(hs, srcp, dstp, zeros_rows)
    return _tc_final(sp, base, degp)


# speedup vs baseline: 1.0813x; 1.0813x over previous
import functools

import jax
import jax.numpy as jnp
from jax import lax
from jax.experimental import pallas as pl
from jax.experimental.pallas import tpu as pltpu
from jax.experimental.pallas import tpu_sc as plsc

N = 10000
D = 128
E = 320000
NCORE = 2
NSUB = 16
NT = NCORE * NSUB
EPT = E // NT
K = 125
NCHT = 80
EPAD = NCHT * K - EPT
UNROLL = 16
NBLK = NCHT // UNROLL
NPAD = 10112
RPT = NPAD // NSUB
TRASH = N

_mesh = plsc.VectorSubcoreMesh(core_axis_name="c", subcore_axis_name="s")


def _deg_body(dst_hbm, zeros_hbm, ones_hbm, out_hbm, dst_v, ones_v, acc_sh):
    c = lax.axis_index("c")
    s = lax.axis_index("s")
    tile = c * NSUB + s
    pltpu.sync_copy(dst_hbm.at[tile], dst_v)
    pltpu.sync_copy(ones_hbm, ones_v)
    pltpu.sync_copy(zeros_hbm, acc_sh.at[pl.ds(s * RPT, RPT)])
    plsc.subcore_barrier()

    def body(j, carry):
        pltpu.sync_copy(ones_v, acc_sh.at[dst_v.at[j]], add=True)
        return carry

    lax.fori_loop(0, NCHT, body, 0)
    plsc.subcore_barrier()
    pltpu.sync_copy(acc_sh.at[pl.ds(s * RPT, RPT)],
                    out_hbm.at[c, pl.ds(s * RPT, RPT)])


_deg_call = functools.partial(
    pl.kernel,
    out_type=jax.ShapeDtypeStruct((NCORE, NPAD, D), jnp.float32),
    mesh=_mesh,
    scratch_types=[
        pltpu.VMEM((NCHT, K), jnp.int32),
        pltpu.VMEM((K, D), jnp.float32),
        pltpu.VMEM_SHARED((NPAD, D), jnp.float32),
    ],
)(_deg_body)


def _agg_body(hs_hbm, src_hbm, dst_hbm, zeros_hbm, out_hbm,
              src_v, dst_v, rows0, rows1, acc_sh, sem0, sem1):
    c = lax.axis_index("c")
    s = lax.axis_index("s")
    tile = c * NSUB + s
    pltpu.sync_copy(zeros_hbm, acc_sh.at[pl.ds(s * RPT, RPT)])
    plsc.subcore_barrier()

    rows = (rows0, rows1)
    sems = (sem0, sem1)
    pltpu.sync_copy(src_hbm.at[tile], src_v)

    def block(b, carry):
        j0 = b * UNROLL
        g_prev = pltpu.async_copy(hs_hbm.at[src_v.at[j0]], rows[0], sems[0])
        pltpu.sync_copy(dst_hbm.at[tile, pl.ds(j0, UNROLL)], dst_v)
        for i in range(1, UNROLL):
            g_cur = pltpu.async_copy(hs_hbm.at[src_v.at[j0 + i]],
                                     rows[i % 2], sems[i % 2])
            g_prev.wait()
            pltpu.sync_copy(rows[(i - 1) % 2], acc_sh.at[dst_v.at[i - 1]],
                            add=True)
            g_prev = g_cur
        g_prev.wait()
        pltpu.sync_copy(rows[(UNROLL - 1) % 2],
                        acc_sh.at[dst_v.at[UNROLL - 1]], add=True)
        return carry

    lax.fori_loop(0, NBLK, block, 0)
    plsc.subcore_barrier()
    pltpu.sync_copy(acc_sh.at[pl.ds(s * RPT, RPT)],
                    out_hbm.at[c, pl.ds(s * RPT, RPT)])


_agg_call = functools.partial(
    pl.kernel,
    out_type=jax.ShapeDtypeStruct((NCORE, NPAD, D), jnp.float32),
    mesh=_mesh,
    scratch_types=[
        pltpu.VMEM((NCHT, K), jnp.int32),
        pltpu.VMEM((UNROLL, K), jnp.int32),
        pltpu.VMEM((K, D), jnp.float32),
        pltpu.VMEM((K, D), jnp.float32),
        pltpu.VMEM_SHARED((NPAD, D), jnp.float32),
        pltpu.SemaphoreType.DMA,
        pltpu.SemaphoreType.DMA,
    ],
)(_agg_body)


_BLK = 2000


def _dis_block(degp):
    deg = degp[0] + degp[1]
    return lax.rsqrt(deg[:, 0:1] + 1.0)


def _tc_first_body(x_ref, degp_ref, wct_ref, wrt_ref, bc_ref, br_ref,
                   hs_ref, base_ref):
    dis = _dis_block(degp_ref[...])
    X = jnp.tanh(x_ref[...])
    h = jnp.dot(X, wct_ref[...], preferred_element_type=jnp.float32)
    hs = h * dis
    r = jnp.dot(h, wrt_ref[...], preferred_element_type=jnp.float32)
    hs_ref[...] = hs
    base_ref[...] = hs * dis + bc_ref[...] - r - br_ref[...]


def _tc_mid_body(sp_ref, basep_ref, degp_ref, wct_ref, wrt_ref, bc_ref, br_ref,
                 hs_ref, base_ref):
    dis = _dis_block(degp_ref[...])
    S = sp_ref[0] + sp_ref[1]
    X = jnp.tanh(S * dis + basep_ref[...])
    h = jnp.dot(X, wct_ref[...], preferred_element_type=jnp.float32)
    hs = h * dis
    r = jnp.dot(h, wrt_ref[...], preferred_element_type=jnp.float32)
    hs_ref[...] = hs
    base_ref[...] = hs * dis + bc_ref[...] - r - br_ref[...]


def _tc_final_body(sp_ref, basep_ref, degp_ref, x_ref):
    dis = _dis_block(degp_ref[...])
    S = sp_ref[0] + sp_ref[1]
    x_ref[...] = jnp.tanh(S * dis + basep_ref[...])


_io_nd = lambda: pl.BlockSpec((_BLK, D), lambda i: (i, 0))
_io_degp = lambda: pl.BlockSpec((NCORE, _BLK, D), lambda i: (0, i, 0))
_io_sp = lambda: pl.BlockSpec((NCORE, _BLK, D), lambda i: (0, i, 0))
_io_w = lambda: pl.BlockSpec((D, D), lambda i: (0, 0))
_io_b = lambda: pl.BlockSpec((1, D), lambda i: (0, 0))

_nd_out = lambda n: [jax.ShapeDtypeStruct((N, D), jnp.float32)] * n

_tc_first = pl.pallas_call(
    _tc_first_body,
    grid=(N // _BLK,),
    in_specs=[_io_nd(), _io_degp(), _io_w(), _io_w(), _io_b(), _io_b()],
    out_specs=[_io_nd(), _io_nd()],
    out_shape=_nd_out(2),
)

_tc_mid = pl.pallas_call(
    _tc_mid_body,
    grid=(N // _BLK,),
    in_specs=[_io_sp(), _io_nd(), _io_degp(), _io_w(), _io_w(), _io_b(), _io_b()],
    out_specs=[_io_nd(), _io_nd()],
    out_shape=_nd_out(2),
)

_tc_final = pl.pallas_call(
    _tc_final_body,
    grid=(N // _BLK,),
    in_specs=[_io_sp(), _io_nd(), _io_degp()],
    out_specs=_io_nd(),
    out_shape=jax.ShapeDtypeStruct((N, D), jnp.float32),
)


def kernel(x, edge_index, W_conv, b_conv, W_res, b_res):
    src = edge_index[0].astype(jnp.int32).reshape(NT, EPT)
    dst = edge_index[1].astype(jnp.int32).reshape(NT, EPT)
    srcp = jnp.pad(src, ((0, 0), (0, EPAD))).reshape(NT, NCHT, K)
    dstp = jnp.pad(dst, ((0, 0), (0, EPAD)),
                   constant_values=TRASH).reshape(NT, NCHT, K)
    wct = W_conv.T
    wrt = W_res.T
    bc = b_conv.reshape(1, D)
    br = b_res.reshape(1, D)
    ones_deg = jnp.ones((K, D), jnp.float32)
    zeros_rows = jnp.zeros((RPT, D), jnp.float32)

    degp = _deg_call(dstp, zeros_rows, ones_deg)
    hs, base = _tc_first(x, degp, wct, wrt, bc, br)
    for _ in range(2):
        sp = _agg_call(hs, srcp, dstp, zeros_rows)
        hs, base = _tc_mid(sp, base, degp, wct, wrt, bc, br)
    sp = _agg_call(hs, srcp, dstp, zeros_rows)
    return _tc_final(sp, base, degp)


# speedup vs baseline: 1.0923x; 1.0102x over previous
import functools

import jax
import jax.numpy as jnp
from jax import lax
from jax.experimental import pallas as pl
from jax.experimental.pallas import tpu as pltpu
from jax.experimental.pallas import tpu_sc as plsc

N = 10000
D = 128
E = 320000
NCORE = 2
NSUB = 16
NT = NCORE * NSUB
EPT = E // NT
K = 125
NCHT = 80
EPAD = NCHT * K - EPT
UNROLL = 40
NBLK = NCHT // UNROLL
NPAD = 10112
RPT = NPAD // NSUB
TRASH = N

_mesh = plsc.VectorSubcoreMesh(core_axis_name="c", subcore_axis_name="s")


def _deg_body(dst_hbm, zeros_hbm, ones_hbm, out_hbm, dst_v, ones_v, acc_sh):
    c = lax.axis_index("c")
    s = lax.axis_index("s")
    tile = c * NSUB + s
    pltpu.sync_copy(dst_hbm.at[tile], dst_v)
    pltpu.sync_copy(ones_hbm, ones_v)
    pltpu.sync_copy(zeros_hbm, acc_sh.at[pl.ds(s * RPT, RPT)])
    plsc.subcore_barrier()

    def body(j, carry):
        pltpu.sync_copy(ones_v, acc_sh.at[dst_v.at[j]], add=True)
        return carry

    lax.fori_loop(0, NCHT, body, 0)
    plsc.subcore_barrier()
    pltpu.sync_copy(acc_sh.at[pl.ds(s * RPT, RPT)],
                    out_hbm.at[c, pl.ds(s * RPT, RPT)])


_deg_call = functools.partial(
    pl.kernel,
    out_type=jax.ShapeDtypeStruct((NCORE, NPAD, D), jnp.float32),
    mesh=_mesh,
    scratch_types=[
        pltpu.VMEM((NCHT, K), jnp.int32),
        pltpu.VMEM((K, D), jnp.float32),
        pltpu.VMEM_SHARED((NPAD, D), jnp.float32),
    ],
)(_deg_body)


def _agg_body(hs_hbm, src_hbm, dst_hbm, zeros_hbm, out_hbm,
              src_v, dst_v, rows0, rows1, acc_sh, sem0, sem1):
    c = lax.axis_index("c")
    s = lax.axis_index("s")
    tile = c * NSUB + s
    pltpu.sync_copy(zeros_hbm, acc_sh.at[pl.ds(s * RPT, RPT)])
    plsc.subcore_barrier()

    rows = (rows0, rows1)
    sems = (sem0, sem1)
    pltpu.sync_copy(src_hbm.at[tile], src_v)

    def block(b, carry):
        j0 = b * UNROLL
        g_prev = pltpu.async_copy(hs_hbm.at[src_v.at[j0]], rows[0], sems[0])
        pltpu.sync_copy(dst_hbm.at[tile, pl.ds(j0, UNROLL)], dst_v)
        for i in range(1, UNROLL):
            g_cur = pltpu.async_copy(hs_hbm.at[src_v.at[j0 + i]],
                                     rows[i % 2], sems[i % 2])
            g_prev.wait()
            pltpu.sync_copy(rows[(i - 1) % 2], acc_sh.at[dst_v.at[i - 1]],
                            add=True)
            g_prev = g_cur
        g_prev.wait()
        pltpu.sync_copy(rows[(UNROLL - 1) % 2],
                        acc_sh.at[dst_v.at[UNROLL - 1]], add=True)
        return carry

    lax.fori_loop(0, NBLK, block, 0)
    plsc.subcore_barrier()
    pltpu.sync_copy(acc_sh.at[pl.ds(s * RPT, RPT)],
                    out_hbm.at[c, pl.ds(s * RPT, RPT)])


_agg_call = functools.partial(
    pl.kernel,
    out_type=jax.ShapeDtypeStruct((NCORE, NPAD, D), jnp.float32),
    mesh=_mesh,
    scratch_types=[
        pltpu.VMEM((NCHT, K), jnp.int32),
        pltpu.VMEM((UNROLL, K), jnp.int32),
        pltpu.VMEM((K, D), jnp.float32),
        pltpu.VMEM((K, D), jnp.float32),
        pltpu.VMEM_SHARED((NPAD, D), jnp.float32),
        pltpu.SemaphoreType.DMA,
        pltpu.SemaphoreType.DMA,
    ],
)(_agg_body)


_BLK = 2000


def _dis_block(degp):
    deg = degp[0] + degp[1]
    return lax.rsqrt(deg[:, 0:1] + 1.0)


def _tc_first_body(x_ref, degp_ref, wct_ref, wrt_ref, bc_ref, br_ref,
                   hs_ref, base_ref):
    dis = _dis_block(degp_ref[...])
    X = jnp.tanh(x_ref[...])
    h = jnp.dot(X, wct_ref[...], preferred_element_type=jnp.float32)
    hs = h * dis
    r = jnp.dot(h, wrt_ref[...], preferred_element_type=jnp.float32)
    hs_ref[...] = hs
    base_ref[...] = hs * dis + bc_ref[...] - r - br_ref[...]


def _tc_mid_body(sp_ref, basep_ref, degp_ref, wct_ref, wrt_ref, bc_ref, br_ref,
                 hs_ref, base_ref):
    dis = _dis_block(degp_ref[...])
    S = sp_ref[0] + sp_ref[1]
    X = jnp.tanh(S * dis + basep_ref[...])
    h = jnp.dot(X, wct_ref[...], preferred_element_type=jnp.float32)
    hs = h * dis
    r = jnp.dot(h, wrt_ref[...], preferred_element_type=jnp.float32)
    hs_ref[...] = hs
    base_ref[...] = hs * dis + bc_ref[...] - r - br_ref[...]


def _tc_final_body(sp_ref, basep_ref, degp_ref, x_ref):
    dis = _dis_block(degp_ref[...])
    S = sp_ref[0] + sp_ref[1]
    x_ref[...] = jnp.tanh(S * dis + basep_ref[...])


_io_nd = lambda: pl.BlockSpec((_BLK, D), lambda i: (i, 0))
_io_degp = lambda: pl.BlockSpec((NCORE, _BLK, D), lambda i: (0, i, 0))
_io_sp = lambda: pl.BlockSpec((NCORE, _BLK, D), lambda i: (0, i, 0))
_io_w = lambda: pl.BlockSpec((D, D), lambda i: (0, 0))
_io_b = lambda: pl.BlockSpec((1, D), lambda i: (0, 0))

_nd_out = lambda n: [jax.ShapeDtypeStruct((N, D), jnp.float32)] * n

_tc_first = pl.pallas_call(
    _tc_first_body,
    grid=(N // _BLK,),
    in_specs=[_io_nd(), _io_degp(), _io_w(), _io_w(), _io_b(), _io_b()],
    out_specs=[_io_nd(), _io_nd()],
    out_shape=_nd_out(2),
)

_tc_mid = pl.pallas_call(
    _tc_mid_body,
    grid=(N // _BLK,),
    in_specs=[_io_sp(), _io_nd(), _io_degp(), _io_w(), _io_w(), _io_b(), _io_b()],
    out_specs=[_io_nd(), _io_nd()],
    out_shape=_nd_out(2),
)

_tc_final = pl.pallas_call(
    _tc_final_body,
    grid=(N // _BLK,),
    in_specs=[_io_sp(), _io_nd(), _io_degp()],
    out_specs=_io_nd(),
    out_shape=jax.ShapeDtypeStruct((N, D), jnp.float32),
)


def kernel(x, edge_index, W_conv, b_conv, W_res, b_res):
    src = edge_index[0].astype(jnp.int32).reshape(NT, EPT)
    dst = edge_index[1].astype(jnp.int32).reshape(NT, EPT)
    srcp = jnp.pad(src, ((0, 0), (0, EPAD))).reshape(NT, NCHT, K)
    dstp = jnp.pad(dst, ((0, 0), (0, EPAD)),
                   constant_values=TRASH).reshape(NT, NCHT, K)
    wct = W_conv.T
    wrt = W_res.T
    bc = b_conv.reshape(1, D)
    br = b_res.reshape(1, D)
    ones_deg = jnp.ones((K, D), jnp.float32)
    zeros_rows = jnp.zeros((RPT, D), jnp.float32)

    degp = _deg_call(dstp, zeros_rows, ones_deg)
    hs, base = _tc_first(x, degp, wct, wrt, bc, br)
    for _ in range(2):
        sp = _agg_call(hs, srcp, dstp, zeros_rows)
        hs, base = _tc_mid(sp, base, degp, wct, wrt, bc, br)
    sp = _agg_call(hs, srcp, dstp, zeros_rows)
    return _tc_final(sp, base, degp)


# speedup vs baseline: 1.1060x; 1.0126x over previous
import functools

import jax
import jax.numpy as jnp
from jax import lax
from jax.experimental import pallas as pl
from jax.experimental.pallas import tpu as pltpu
from jax.experimental.pallas import tpu_sc as plsc

N = 10000
D = 128
E = 320000
NCORE = 2
NSUB = 16
NT = NCORE * NSUB
EPT = E // NT
K = 125
NCHT = 80
EPAD = NCHT * K - EPT
UNROLL = 40
NBLK = NCHT // UNROLL
NPAD = 10112
RPT = NPAD // NSUB
TRASH = N

_mesh = plsc.VectorSubcoreMesh(core_axis_name="c", subcore_axis_name="s")


def _deg_body(dst_hbm, zeros_hbm, ones_hbm, out_hbm, dst_v, ones_v, acc_sh):
    c = lax.axis_index("c")
    s = lax.axis_index("s")
    tile = c * NSUB + s
    pltpu.sync_copy(dst_hbm.at[tile], dst_v)
    pltpu.sync_copy(ones_hbm, ones_v)
    pltpu.sync_copy(zeros_hbm, acc_sh.at[pl.ds(s * RPT, RPT)])
    plsc.subcore_barrier()

    def body(j, carry):
        pltpu.sync_copy(ones_v, acc_sh.at[dst_v.at[j]], add=True)
        return carry

    lax.fori_loop(0, NCHT, body, 0)
    plsc.subcore_barrier()
    pltpu.sync_copy(acc_sh.at[pl.ds(s * RPT, RPT)],
                    out_hbm.at[c, pl.ds(s * RPT, RPT)])


_deg_call = functools.partial(
    pl.kernel,
    out_type=jax.ShapeDtypeStruct((NCORE, NPAD, D), jnp.float32),
    mesh=_mesh,
    scratch_types=[
        pltpu.VMEM((NCHT, K), jnp.int32),
        pltpu.VMEM((K, D), jnp.float32),
        pltpu.VMEM_SHARED((NPAD, D), jnp.float32),
    ],
)(_deg_body)


def _agg_body(hs_hbm, src_hbm, dst_hbm, zeros_hbm, out_hbm,
              src_v, dst_v, rows0, rows1, acc_sh, sem0, sem1):
    c = lax.axis_index("c")
    s = lax.axis_index("s")
    tile = c * NSUB + s
    pltpu.sync_copy(zeros_hbm, acc_sh.at[pl.ds(s * RPT, RPT)])
    plsc.subcore_barrier()

    rows = (rows0, rows1)
    sems = (sem0, sem1)
    pltpu.sync_copy(src_hbm.at[tile], src_v)

    def block(b, carry):
        j0 = b * UNROLL
        g_prev = pltpu.async_copy(hs_hbm.at[src_v.at[j0]], rows[0], sems[0])
        pltpu.sync_copy(dst_hbm.at[tile, pl.ds(j0, UNROLL)], dst_v)
        for i in range(1, UNROLL):
            g_cur = pltpu.async_copy(hs_hbm.at[src_v.at[j0 + i]],
                                     rows[i % 2], sems[i % 2])
            g_prev.wait()
            pltpu.sync_copy(rows[(i - 1) % 2], acc_sh.at[dst_v.at[i - 1]],
                            add=True)
            g_prev = g_cur
        g_prev.wait()
        pltpu.sync_copy(rows[(UNROLL - 1) % 2],
                        acc_sh.at[dst_v.at[UNROLL - 1]], add=True)
        return carry

    lax.fori_loop(0, NBLK, block, 0)
    plsc.subcore_barrier()
    pltpu.sync_copy(acc_sh.at[pl.ds(s * RPT, RPT)],
                    out_hbm.at[c, pl.ds(s * RPT, RPT)])


_agg_call = functools.partial(
    pl.kernel,
    out_type=jax.ShapeDtypeStruct((NCORE, NPAD, D), jnp.float32),
    mesh=_mesh,
    scratch_types=[
        pltpu.VMEM((NCHT, K), jnp.int32),
        pltpu.VMEM((UNROLL, K), jnp.int32),
        pltpu.VMEM((K, D), jnp.float32),
        pltpu.VMEM((K, D), jnp.float32),
        pltpu.VMEM_SHARED((NPAD, D), jnp.float32),
        pltpu.SemaphoreType.DMA,
        pltpu.SemaphoreType.DMA,
    ],
)(_agg_body)


_BLK = 5000


def _dis_block(degp):
    deg = degp[0] + degp[1]
    return lax.rsqrt(deg[:, 0:1] + 1.0)


def _tc_first_body(x_ref, degp_ref, wct_ref, wrt_ref, bc_ref, br_ref,
                   hs_ref, base_ref, dis_ref):
    dis = _dis_block(degp_ref[...])
    dis_ref[...] = jnp.broadcast_to(dis, (_BLK, D))
    X = jnp.tanh(x_ref[...])
    h = jnp.dot(X, wct_ref[...], preferred_element_type=jnp.float32)
    hs = h * dis
    r = jnp.dot(h, wrt_ref[...], preferred_element_type=jnp.float32)
    hs_ref[...] = hs
    base_ref[...] = hs * dis + bc_ref[...] - r - br_ref[...]


def _tc_mid_body(sp_ref, basep_ref, dis_in_ref, wct_ref, wrt_ref, bc_ref, br_ref,
                 hs_ref, base_ref):
    dis = dis_in_ref[...]
    S = sp_ref[0] + sp_ref[1]
    X = jnp.tanh(S * dis + basep_ref[...])
    h = jnp.dot(X, wct_ref[...], preferred_element_type=jnp.float32)
    hs = h * dis
    r = jnp.dot(h, wrt_ref[...], preferred_element_type=jnp.float32)
    hs_ref[...] = hs
    base_ref[...] = hs * dis + bc_ref[...] - r - br_ref[...]


def _tc_final_body(sp_ref, basep_ref, dis_in_ref, x_ref):
    dis = dis_in_ref[...]
    S = sp_ref[0] + sp_ref[1]
    x_ref[...] = jnp.tanh(S * dis + basep_ref[...])


_io_nd = lambda: pl.BlockSpec((_BLK, D), lambda i: (i, 0))
_io_degp = lambda: pl.BlockSpec((NCORE, _BLK, D), lambda i: (0, i, 0))
_io_sp = lambda: pl.BlockSpec((NCORE, _BLK, D), lambda i: (0, i, 0))
_io_w = lambda: pl.BlockSpec((D, D), lambda i: (0, 0))
_io_b = lambda: pl.BlockSpec((1, D), lambda i: (0, 0))

_nd_out = lambda n: [jax.ShapeDtypeStruct((N, D), jnp.float32)] * n

_tc_first = pl.pallas_call(
    _tc_first_body,
    grid=(N // _BLK,),
    in_specs=[_io_nd(), _io_degp(), _io_w(), _io_w(), _io_b(), _io_b()],
    out_specs=[_io_nd(), _io_nd(), _io_nd()],
    out_shape=_nd_out(3),
)

_tc_mid = pl.pallas_call(
    _tc_mid_body,
    grid=(N // _BLK,),
    in_specs=[_io_sp(), _io_nd(), _io_nd(), _io_w(), _io_w(), _io_b(), _io_b()],
    out_specs=[_io_nd(), _io_nd()],
    out_shape=_nd_out(2),
)

_tc_final = pl.pallas_call(
    _tc_final_body,
    grid=(N // _BLK,),
    in_specs=[_io_sp(), _io_nd(), _io_nd()],
    out_specs=_io_nd(),
    out_shape=jax.ShapeDtypeStruct((N, D), jnp.float32),
)


def kernel(x, edge_index, W_conv, b_conv, W_res, b_res):
    src = edge_index[0].astype(jnp.int32).reshape(NT, EPT)
    dst = edge_index[1].astype(jnp.int32).reshape(NT, EPT)
    srcp = jnp.pad(src, ((0, 0), (0, EPAD))).reshape(NT, NCHT, K)
    dstp = jnp.pad(dst, ((0, 0), (0, EPAD)),
                   constant_values=TRASH).reshape(NT, NCHT, K)
    wct = W_conv.T
    wrt = W_res.T
    bc = b_conv.reshape(1, D)
    br = b_res.reshape(1, D)
    ones_deg = jnp.ones((K, D), jnp.float32)
    zeros_rows = jnp.zeros((RPT, D), jnp.float32)

    degp = _deg_call(dstp, zeros_rows, ones_deg)
    hs, base, dis = _tc_first(x, degp, wct, wrt, bc, br)
    for _ in range(2):
        sp = _agg_call(hs, srcp, dstp, zeros_rows)
        hs, base = _tc_mid(sp, base, dis, wct, wrt, bc, br)
    sp = _agg_call(hs, srcp, dstp, zeros_rows)
    return _tc_final(sp, base, dis)


# speedup vs baseline: 1.2208x; 1.1038x over previous
import functools

import jax
import jax.numpy as jnp
from jax import lax
from jax.experimental import pallas as pl
from jax.experimental.pallas import tpu as pltpu
from jax.experimental.pallas import tpu_sc as plsc

N = 10000
D = 128
E = 320000
NCORE = 2
NSUB = 16
NT = NCORE * NSUB
EPT = E // NT
K = 125
NCHT = 80
EPAD = NCHT * K - EPT
UNROLL = 40
NBLK = NCHT // UNROLL
NPAD = 10112
RPT = NPAD // NSUB
TRASH = N

_mesh = plsc.VectorSubcoreMesh(core_axis_name="c", subcore_axis_name="s")


DEGW = 32
DEGK = 128


def _deg_body(dst_hbm, zeros_hbm, out_hbm, dst_v, ones_v, acc_sh):
    c = lax.axis_index("c")
    s = lax.axis_index("s")
    tile = c * NSUB + s
    pltpu.sync_copy(dst_hbm.at[tile], dst_v)

    ones16 = jnp.ones((16,), jnp.float32)

    def fill(i, carry):
        ones_v[i, pl.ds(0, 16)] = ones16
        ones_v[i, pl.ds(16, 16)] = ones16
        return carry

    lax.fori_loop(0, DEGK, fill, 0)
    pltpu.sync_copy(zeros_hbm, acc_sh.at[pl.ds(s * RPT, RPT)])
    plsc.subcore_barrier()

    def body(j, carry):
        pltpu.sync_copy(ones_v, acc_sh.at[dst_v.at[j]], add=True)
        return carry

    lax.fori_loop(0, NCHT, body, 0)
    plsc.subcore_barrier()
    pltpu.sync_copy(acc_sh.at[pl.ds(s * RPT, RPT)],
                    out_hbm.at[c, pl.ds(s * RPT, RPT), pl.ds(0, DEGW)])


_deg_call = functools.partial(
    pl.kernel,
    out_type=jax.ShapeDtypeStruct((NCORE, NPAD, D), jnp.float32),
    mesh=_mesh,
    scratch_types=[
        pltpu.VMEM((NCHT, DEGK), jnp.int32),
        pltpu.VMEM((DEGK, DEGW), jnp.float32),
        pltpu.VMEM_SHARED((NPAD, DEGW), jnp.float32),
    ],
    compiler_params=pltpu.CompilerParams(use_tc_tiling_on_sc=False),
)(_deg_body)


def _agg_body(hs_hbm, src_hbm, dst_hbm, zeros_hbm, out_hbm,
              src_v, dst_v, rows0, rows1, acc_sh, sem0, sem1):
    c = lax.axis_index("c")
    s = lax.axis_index("s")
    tile = c * NSUB + s
    pltpu.sync_copy(zeros_hbm, acc_sh.at[pl.ds(s * RPT, RPT)])
    plsc.subcore_barrier()

    rows = (rows0, rows1)
    sems = (sem0, sem1)
    pltpu.sync_copy(src_hbm.at[tile], src_v)

    def block(b, carry):
        j0 = b * UNROLL
        g_prev = pltpu.async_copy(hs_hbm.at[src_v.at[j0]], rows[0], sems[0])
        pltpu.sync_copy(dst_hbm.at[tile, pl.ds(j0, UNROLL)], dst_v)
        for i in range(1, UNROLL):
            g_cur = pltpu.async_copy(hs_hbm.at[src_v.at[j0 + i]],
                                     rows[i % 2], sems[i % 2])
            g_prev.wait()
            pltpu.sync_copy(rows[(i - 1) % 2], acc_sh.at[dst_v.at[i - 1]],
                            add=True)
            g_prev = g_cur
        g_prev.wait()
        pltpu.sync_copy(rows[(UNROLL - 1) % 2],
                        acc_sh.at[dst_v.at[UNROLL - 1]], add=True)
        return carry

    lax.fori_loop(0, NBLK, block, 0)
    plsc.subcore_barrier()
    pltpu.sync_copy(acc_sh.at[pl.ds(s * RPT, RPT)],
                    out_hbm.at[c, pl.ds(s * RPT, RPT)])


_agg_call = functools.partial(
    pl.kernel,
    out_type=jax.ShapeDtypeStruct((NCORE, NPAD, D), jnp.float32),
    mesh=_mesh,
    scratch_types=[
        pltpu.VMEM((NCHT, K), jnp.int32),
        pltpu.VMEM((UNROLL, K), jnp.int32),
        pltpu.VMEM((K, D), jnp.float32),
        pltpu.VMEM((K, D), jnp.float32),
        pltpu.VMEM_SHARED((NPAD, D), jnp.float32),
        pltpu.SemaphoreType.DMA,
        pltpu.SemaphoreType.DMA,
    ],
)(_agg_body)


_BLK = 5000


def _dis_block(degp):
    deg = degp[0] + degp[1]
    return lax.rsqrt(deg[:, 0:1] + 1.0)


def _tc_first_body(x_ref, degp_ref, wct_ref, wrt_ref, bc_ref, br_ref,
                   hs_ref, base_ref, dis_ref):
    dis = _dis_block(degp_ref[...])
    dis_ref[...] = jnp.broadcast_to(dis, (_BLK, D))
    X = jnp.tanh(x_ref[...])
    h = jnp.dot(X, wct_ref[...], preferred_element_type=jnp.float32)
    hs = h * dis
    r = jnp.dot(h, wrt_ref[...], preferred_element_type=jnp.float32)
    hs_ref[...] = hs
    base_ref[...] = hs * dis + bc_ref[...] - r - br_ref[...]


def _tc_mid_body(sp_ref, basep_ref, dis_in_ref, wct_ref, wrt_ref, bc_ref, br_ref,
                 hs_ref, base_ref):
    dis = dis_in_ref[...]
    S = sp_ref[0] + sp_ref[1]
    X = jnp.tanh(S * dis + basep_ref[...])
    h = jnp.dot(X, wct_ref[...], preferred_element_type=jnp.float32)
    hs = h * dis
    r = jnp.dot(h, wrt_ref[...], preferred_element_type=jnp.float32)
    hs_ref[...] = hs
    base_ref[...] = hs * dis + bc_ref[...] - r - br_ref[...]


def _tc_final_body(sp_ref, basep_ref, dis_in_ref, x_ref):
    dis = dis_in_ref[...]
    S = sp_ref[0] + sp_ref[1]
    x_ref[...] = jnp.tanh(S * dis + basep_ref[...])


_io_nd = lambda: pl.BlockSpec((_BLK, D), lambda i: (i, 0))
_io_degp = lambda: pl.BlockSpec((NCORE, _BLK, D), lambda i: (0, i, 0))
_io_sp = lambda: pl.BlockSpec((NCORE, _BLK, D), lambda i: (0, i, 0))
_io_w = lambda: pl.BlockSpec((D, D), lambda i: (0, 0))
_io_b = lambda: pl.BlockSpec((1, D), lambda i: (0, 0))

_nd_out = lambda n: [jax.ShapeDtypeStruct((N, D), jnp.float32)] * n

_tc_first = pl.pallas_call(
    _tc_first_body,
    grid=(N // _BLK,),
    in_specs=[_io_nd(), _io_degp(), _io_w(), _io_w(), _io_b(), _io_b()],
    out_specs=[_io_nd(), _io_nd(), _io_nd()],
    out_shape=_nd_out(3),
)

_tc_mid = pl.pallas_call(
    _tc_mid_body,
    grid=(N // _BLK,),
    in_specs=[_io_sp(), _io_nd(), _io_nd(), _io_w(), _io_w(), _io_b(), _io_b()],
    out_specs=[_io_nd(), _io_nd()],
    out_shape=_nd_out(2),
)

_tc_final = pl.pallas_call(
    _tc_final_body,
    grid=(N // _BLK,),
    in_specs=[_io_sp(), _io_nd(), _io_nd()],
    out_specs=_io_nd(),
    out_shape=jax.ShapeDtypeStruct((N, D), jnp.float32),
)


def kernel(x, edge_index, W_conv, b_conv, W_res, b_res):
    src = edge_index[0].astype(jnp.int32).reshape(NT, EPT)
    dst = edge_index[1].astype(jnp.int32).reshape(NT, EPT)
    srcp = jnp.pad(src, ((0, 0), (0, EPAD))).reshape(NT, NCHT, K)
    dstp = jnp.pad(dst, ((0, 0), (0, EPAD)),
                   constant_values=TRASH).reshape(NT, NCHT, K)
    wct = W_conv.T
    wrt = W_res.T
    bc = b_conv.reshape(1, D)
    br = b_res.reshape(1, D)
    zeros_rows = jnp.zeros((RPT, D), jnp.float32)
    zeros_deg = jnp.zeros((RPT, DEGW), jnp.float32)
    dst_deg = jnp.concatenate(
        [dst, jnp.full((NT, NCHT * DEGK - EPT), TRASH, jnp.int32)],
        axis=1).reshape(NT, NCHT, DEGK)

    degp = _deg_call(dst_deg, zeros_deg)
    hs, base, dis = _tc_first(x, degp, wct, wrt, bc, br)
    for _ in range(2):
        sp = _agg_call(hs, srcp, dstp, zeros_rows)
        hs, base = _tc_mid(sp, base, dis, wct, wrt, bc, br)
    sp = _agg_call(hs, srcp, dstp, zeros_rows)
    return _tc_final(sp, base, dis)


# speedup vs baseline: 1.2439x; 1.0189x over previous
import functools

import jax
import jax.numpy as jnp
from jax import lax
from jax.experimental import pallas as pl
from jax.experimental.pallas import tpu as pltpu
from jax.experimental.pallas import tpu_sc as plsc

N = 10000
D = 128
E = 320000
NCORE = 2
NSUB = 16
NT = NCORE * NSUB
EPT = E // NT
K = 125
NCHT = 80
EPAD = NCHT * K - EPT
UNROLL = 40
NBLK = NCHT // UNROLL
NPAD = 10112
RPT = NPAD // NSUB
TRASH = N

_mesh = plsc.VectorSubcoreMesh(core_axis_name="c", subcore_axis_name="s")


DEGW = 16
DEGK = 128


def _deg_body(dst_hbm, zeros_hbm, out_hbm, dst_v, ones_v, acc_sh):
    c = lax.axis_index("c")
    s = lax.axis_index("s")
    tile = c * NSUB + s
    pltpu.sync_copy(dst_hbm.at[tile], dst_v)

    ones16 = jnp.ones((16,), jnp.float32)

    def fill(i, carry):
        ones_v[i, pl.ds(0, 16)] = ones16
        return carry

    lax.fori_loop(0, DEGK, fill, 0)
    pltpu.sync_copy(zeros_hbm, acc_sh.at[pl.ds(s * RPT, RPT)])
    plsc.subcore_barrier()

    def body(j, carry):
        pltpu.sync_copy(ones_v, acc_sh.at[dst_v.at[j]], add=True)
        return carry

    lax.fori_loop(0, NCHT, body, 0)
    plsc.subcore_barrier()
    pltpu.sync_copy(acc_sh.at[pl.ds(s * RPT, RPT)],
                    out_hbm.at[c, pl.ds(s * RPT, RPT), pl.ds(0, DEGW)])


_deg_call = functools.partial(
    pl.kernel,
    out_type=jax.ShapeDtypeStruct((NCORE, NPAD, D), jnp.float32),
    mesh=_mesh,
    scratch_types=[
        pltpu.VMEM((NCHT, DEGK), jnp.int32),
        pltpu.VMEM((DEGK, DEGW), jnp.float32),
        pltpu.VMEM_SHARED((NPAD, DEGW), jnp.float32),
    ],
    compiler_params=pltpu.CompilerParams(use_tc_tiling_on_sc=False),
)(_deg_body)


def _agg_body(hs_hbm, src_hbm, dst_hbm, zeros_hbm, out_hbm,
              src_v, dst_v, rows0, rows1, acc_sh, sem0, sem1):
    c = lax.axis_index("c")
    s = lax.axis_index("s")
    tile = c * NSUB + s
    pltpu.sync_copy(zeros_hbm, acc_sh.at[pl.ds(s * RPT, RPT)])
    plsc.subcore_barrier()

    rows = (rows0, rows1)
    sems = (sem0, sem1)
    pltpu.sync_copy(src_hbm.at[tile], src_v)

    def block(b, carry):
        j0 = b * UNROLL
        g_prev = pltpu.async_copy(hs_hbm.at[src_v.at[j0]], rows[0], sems[0])
        pltpu.sync_copy(dst_hbm.at[tile, pl.ds(j0, UNROLL)], dst_v)
        for i in range(1, UNROLL):
            g_cur = pltpu.async_copy(hs_hbm.at[src_v.at[j0 + i]],
                                     rows[i % 2], sems[i % 2])
            g_prev.wait()
            pltpu.sync_copy(rows[(i - 1) % 2], acc_sh.at[dst_v.at[i - 1]],
                            add=True)
            g_prev = g_cur
        g_prev.wait()
        pltpu.sync_copy(rows[(UNROLL - 1) % 2],
                        acc_sh.at[dst_v.at[UNROLL - 1]], add=True)
        return carry

    lax.fori_loop(0, NBLK, block, 0)
    plsc.subcore_barrier()
    pltpu.sync_copy(acc_sh.at[pl.ds(s * RPT, RPT)],
                    out_hbm.at[c, pl.ds(s * RPT, RPT)])


_agg_call = functools.partial(
    pl.kernel,
    out_type=jax.ShapeDtypeStruct((NCORE, NPAD, D), jnp.float32),
    mesh=_mesh,
    scratch_types=[
        pltpu.VMEM((NCHT, K), jnp.int32),
        pltpu.VMEM((UNROLL, K), jnp.int32),
        pltpu.VMEM((K, D), jnp.float32),
        pltpu.VMEM((K, D), jnp.float32),
        pltpu.VMEM_SHARED((NPAD, D), jnp.float32),
        pltpu.SemaphoreType.DMA,
        pltpu.SemaphoreType.DMA,
    ],
)(_agg_body)


_BLK = 5000


def _dis_block(degp):
    deg = degp[0] + degp[1]
    return lax.rsqrt(deg[:, 0:1] + 1.0)


def _tc_first_body(x_ref, degp_ref, wct_ref, wrt_ref, bc_ref, br_ref,
                   hs_ref, base_ref, dis_ref):
    dis = _dis_block(degp_ref[...])
    dis_ref[...] = jnp.broadcast_to(dis, (_BLK, D))
    X = jnp.tanh(x_ref[...])
    h = jnp.dot(X, wct_ref[...], preferred_element_type=jnp.float32)
    hs = h * dis
    r = jnp.dot(h, wrt_ref[...], preferred_element_type=jnp.float32)
    hs_ref[...] = hs
    base_ref[...] = hs * dis + bc_ref[...] - r - br_ref[...]


def _tc_mid_body(sp_ref, basep_ref, dis_in_ref, wct_ref, wrt_ref, bc_ref, br_ref,
                 hs_ref, base_ref):
    dis = dis_in_ref[...]
    S = sp_ref[0] + sp_ref[1]
    X = jnp.tanh(S * dis + basep_ref[...])
    h = jnp.dot(X, wct_ref[...], preferred_element_type=jnp.float32)
    hs = h * dis
    r = jnp.dot(h, wrt_ref[...], preferred_element_type=jnp.float32)
    hs_ref[...] = hs
    base_ref[...] = hs * dis + bc_ref[...] - r - br_ref[...]


def _tc_final_body(sp_ref, basep_ref, dis_in_ref, x_ref):
    dis = dis_in_ref[...]
    S = sp_ref[0] + sp_ref[1]
    x_ref[...] = jnp.tanh(S * dis + basep_ref[...])


_io_nd = lambda: pl.BlockSpec((_BLK, D), lambda i: (i, 0))
_io_degp = lambda: pl.BlockSpec((NCORE, _BLK, D), lambda i: (0, i, 0))
_io_sp = lambda: pl.BlockSpec((NCORE, _BLK, D), lambda i: (0, i, 0))
_io_w = lambda: pl.BlockSpec((D, D), lambda i: (0, 0))
_io_b = lambda: pl.BlockSpec((1, D), lambda i: (0, 0))

_nd_out = lambda n: [jax.ShapeDtypeStruct((N, D), jnp.float32)] * n

_tc_first = pl.pallas_call(
    _tc_first_body,
    grid=(N // _BLK,),
    in_specs=[_io_nd(), _io_degp(), _io_w(), _io_w(), _io_b(), _io_b()],
    out_specs=[_io_nd(), _io_nd(), _io_nd()],
    out_shape=_nd_out(3),
)

_tc_mid = pl.pallas_call(
    _tc_mid_body,
    grid=(N // _BLK,),
    in_specs=[_io_sp(), _io_nd(), _io_nd(), _io_w(), _io_w(), _io_b(), _io_b()],
    out_specs=[_io_nd(), _io_nd()],
    out_shape=_nd_out(2),
)

_tc_final = pl.pallas_call(
    _tc_final_body,
    grid=(N // _BLK,),
    in_specs=[_io_sp(), _io_nd(), _io_nd()],
    out_specs=_io_nd(),
    out_shape=jax.ShapeDtypeStruct((N, D), jnp.float32),
)


def kernel(x, edge_index, W_conv, b_conv, W_res, b_res):
    src = edge_index[0].astype(jnp.int32).reshape(NT, EPT)
    dst = edge_index[1].astype(jnp.int32).reshape(NT, EPT)
    srcp = jnp.pad(src, ((0, 0), (0, EPAD))).reshape(NT, NCHT, K)
    dstp = jnp.pad(dst, ((0, 0), (0, EPAD)),
                   constant_values=TRASH).reshape(NT, NCHT, K)
    wct = W_conv.T
    wrt = W_res.T
    bc = b_conv.reshape(1, D)
    br = b_res.reshape(1, D)
    zeros_rows = jnp.zeros((RPT, D), jnp.float32)
    zeros_deg = jnp.zeros((RPT, DEGW), jnp.float32)
    dst_deg = jnp.concatenate(
        [dst, jnp.full((NT, NCHT * DEGK - EPT), TRASH, jnp.int32)],
        axis=1).reshape(NT, NCHT, DEGK)

    degp = _deg_call(dst_deg, zeros_deg)
    hs, base, dis = _tc_first(x, degp, wct, wrt, bc, br)
    for _ in range(2):
        sp = _agg_call(hs, srcp, dstp, zeros_rows)
        hs, base = _tc_mid(sp, base, dis, wct, wrt, bc, br)
    sp = _agg_call(hs, srcp, dstp, zeros_rows)
    return _tc_final(sp, base, dis)
